# Initial kernel scaffold; baseline (speedup 1.0000x reference)
import jax
import jax.numpy as jnp
from jax.experimental import pallas as pl


def kernel(x, edge_idx, edge_attr, W, att_src, att_dst, W_edge, att_edge, bias, gamma, beta):
    raise NotImplementedError("write your pallas kernel here")


# speedup vs baseline: 107.3440x; 107.3440x over previous
import functools

import jax
import jax.numpy as jnp
from jax import lax
from jax.experimental import pallas as pl
from jax.experimental.pallas import tpu as pltpu
from jax.experimental.pallas import tpu_sc as plsc

N = 10000
E = 320000
F_IN = 128
H = 8
C = 8
HC = H * C
HSP = 72
ED = 2

_NC = 2
_NS = 16
_NW = _NC * _NS
_EPW = E // _NW
_K = 80
_NCHUNK = _EPW // _K
_NP = 10240
_NPT = _NP // _NS
_ZR = 64


def _lrelu(v, s):
    return jnp.where(v >= 0, v, s * v)


def _prep_body(x_ref, w_ref, as_ref, ad_ref, eat_ref, we_ref, ae_ref,
               hs_out, adt_out, m2_out, aself_out):
    wa_cols = []
    wd_cols = []
    for hh in range(H):
        blk = w_ref[:, hh * C:(hh + 1) * C]
        wa_cols.append((blk * as_ref[hh:hh + 1, :]).sum(axis=1, keepdims=True))
        wd_cols.append((blk * ad_ref[hh:hh + 1, :]).sum(axis=1, keepdims=True))
    w80 = jnp.concatenate([w_ref[...]] + wa_cols + wd_cols, axis=1)
    m = jnp.dot(x_ref[...], w80, preferred_element_type=jnp.float32)
    hs_out[pl.ds(0, N), :] = m[:, 0:HSP]
    hs_out[pl.ds(N, _NP - N), :] = jnp.zeros((_NP - N, HSP), jnp.float32)
    adt_out[pl.ds(0, N), :] = m[:, HC + H:HC + 2 * H]
    adt_out[pl.ds(N, _NP - N), :] = jnp.zeros((_NP - N, H), jnp.float32)
    m2_cols = []
    for hh in range(H):
        m2_cols.append((we_ref[:, hh * C:(hh + 1) * C]
                        * ae_ref[hh:hh + 1, :]).sum(axis=1, keepdims=True))
    m2 = jnp.concatenate(m2_cols, axis=1)
    m2_out[...] = m2
    ea_mean = eat_ref[...].mean(axis=1, keepdims=True)
    aself_out[...] = (ea_mean[0:1, 0:1] * m2[0:1, :]
                      + ea_mean[1:2, 0:1] * m2[1:2, :])


def _tc_prep(x, W, att_src, att_dst, eaT, W_edge, att_edge):
    return pl.pallas_call(
        _prep_body,
        out_shape=[
            jax.ShapeDtypeStruct((_NP, HSP), jnp.float32),
            jax.ShapeDtypeStruct((_NP, H), jnp.float32),
            jax.ShapeDtypeStruct((ED, H), jnp.float32),
            jax.ShapeDtypeStruct((1, H), jnp.float32),
        ],
    )(x, W, att_src, att_dst, eaT, W_edge, att_edge)


def _sc_edge_call(src, dst, ea0, ea1, m2, hs, ad):
    mesh = plsc.VectorSubcoreMesh(core_axis_name="c", subcore_axis_name="s",
                                  num_cores=_NC, num_subcores=_NS)

    @functools.partial(
        pl.kernel,
        out_type=[
            jax.ShapeDtypeStruct((_NC, _NP, HC), jnp.float32),
            jax.ShapeDtypeStruct((_NC, _NP, H), jnp.float32),
        ],
        mesh=mesh,
        compiler_params=pltpu.CompilerParams(needs_layout_passes=False, use_tc_tiling_on_sc=False),
        scratch_types=[
            pltpu.VMEM((_K,), jnp.int32),
            pltpu.VMEM((_K,), jnp.int32),
            pltpu.VMEM((_K,), jnp.float32),
            pltpu.VMEM((_K,), jnp.float32),
            pltpu.VMEM((ED, H), jnp.float32),
            pltpu.VMEM((_K, H), jnp.float32),
            pltpu.VMEM((_K, H), jnp.float32),
            pltpu.VMEM((_K, HSP), jnp.float32),
            pltpu.VMEM((_K, HC), jnp.float32),
            pltpu.VMEM((_ZR, HC), jnp.float32),
            pltpu.VMEM((_ZR, H), jnp.float32),
            pltpu.VMEM_SHARED((_NP, HC), jnp.float32),
            pltpu.VMEM_SHARED((_NP, H), jnp.float32),
            pltpu.SemaphoreType.DMA,
            pltpu.SemaphoreType.DMA,
            pltpu.SemaphoreType.DMA,
        ],
    )
    def kern(src_hbm, dst_hbm, ea0_hbm, ea1_hbm, m2_hbm, hs_hbm, ad_hbm,
             msg_out, den_out,
             srcv, dstv, ea0v, ea1v, m2v, adv, exv, hsv, msgv, zbuf64, zbuf8,
             smsg, sden, sem_h, sem_d, sem_l):
        cid = lax.axis_index("c")
        sid = lax.axis_index("s")
        wid = sid * _NC + cid

        i16 = lax.iota(jnp.int32, 16)
        i_div8 = i16 // 8
        i_mod8 = lax.rem(i16, 8)
        zeros16 = jnp.zeros((16,), jnp.float32)
        zeros16i = jnp.zeros((16,), jnp.int32)
        row0 = sid * _NPT

        pltpu.sync_copy(m2_hbm, m2v)
        m0 = plsc.load_gather(m2v, [zeros16i, i_mod8])
        m1 = plsc.load_gather(m2v, [zeros16i + 1, i_mod8])

        def _zfill(i, _):
            for q in range(HC // 16):
                zbuf64[i, pl.ds(16 * q, 16)] = zeros16
            return 0
        lax.fori_loop(0, _ZR, _zfill, 0)

        def _zfill8(i, _):
            rows = jnp.full((16,), 2 * i, jnp.int32) + i_div8
            plsc.store_scatter(zbuf8, [rows, i_mod8], zeros16)
            return 0
        lax.fori_loop(0, _ZR // 2, _zfill8, 0)

        for j in range(_NPT // _ZR):
            r = row0 + j * _ZR
            pltpu.sync_copy(zbuf64, smsg.at[pl.ds(r, _ZR)])
            pltpu.sync_copy(zbuf8, sden.at[pl.ds(r, _ZR)])
        plsc.subcore_barrier()

        ebase = wid * _EPW

        def _chunk(ci, _):
            base = pl.multiple_of(ebase + ci * _K, 8)
            cp1 = pltpu.async_copy(src_hbm.at[pl.ds(base, _K)], srcv, sem_l)
            cp2 = pltpu.async_copy(dst_hbm.at[pl.ds(base, _K)], dstv, sem_l)
            cp3 = pltpu.async_copy(ea0_hbm.at[pl.ds(base, _K)], ea0v, sem_l)
            cp4 = pltpu.async_copy(ea1_hbm.at[pl.ds(base, _K)], ea1v, sem_l)
            cp1.wait()
            cp2.wait()
            cp_h = pltpu.async_copy(hs_hbm.at[srcv], hsv, sem_h)
            cp_d = pltpu.async_copy(ad_hbm.at[dstv], adv, sem_d)
            cp3.wait()
            cp4.wait()
            cp_h.wait()
            cp_d.wait()

            @plsc.parallel_loop(0, _K // 2, unroll=2)
            def _ex(j):
                rows = jnp.full((16,), 2 * j, jnp.int32) + i_div8
                cols_s = jnp.full((16,), HC, jnp.int32) + i_mod8
                e0 = plsc.load_gather(ea0v, [rows])
                e1 = plsc.load_gather(ea1v, [rows])
                a = (plsc.load_gather(hsv, [rows, cols_s])
                     + plsc.load_gather(adv, [rows, i_mod8])
                     + e0 * m0 + e1 * m1)
                e = jnp.exp(jnp.where(a >= 0, a, 0.2 * a))
                plsc.store_scatter(exv, [rows, i_mod8], e)

            @plsc.parallel_loop(0, _K, unroll=2)
            def _mul(k):
                krow = jnp.full((16,), k, jnp.int32)
                for q in range(HC // 16):
                    eb = plsc.load_gather(exv, [krow, 2 * q + i_div8])
                    msgv[k, pl.ds(16 * q, 16)] = hsv[k, pl.ds(16 * q, 16)] * eb

            pltpu.sync_copy(exv, sden.at[dstv], add=True)
            pltpu.sync_copy(msgv, smsg.at[dstv], add=True)
            return 0

        lax.fori_loop(0, _NCHUNK, _chunk, 0)
        plsc.subcore_barrier()

        for j in range(_NPT // _ZR):
            r = row0 + j * _ZR
            pltpu.sync_copy(smsg.at[pl.ds(r, _ZR)], zbuf64)
            pltpu.sync_copy(zbuf64, msg_out.at[cid, pl.ds(r, _ZR)])
            pltpu.sync_copy(sden.at[pl.ds(r, _ZR)], zbuf8)
            pltpu.sync_copy(zbuf8, den_out.at[cid, pl.ds(r, _ZR)])

    return kern(src, dst, ea0, ea1, m2, hs, ad)


def _final_body(msg_ref, den_ref, hs_ref, ad_ref, aself_ref,
                bias_ref, gamma_ref, beta_ref, y_out):
    r72 = jax.lax.broadcasted_iota(jnp.int32, (HSP, HC), 0)
    c72 = jax.lax.broadcasted_iota(jnp.int32, (HSP, HC), 1)
    ps = jnp.where(r72 == HC + c72 // C, 1.0, 0.0)
    r8 = jax.lax.broadcasted_iota(jnp.int32, (H, HC), 0)
    c8 = jax.lax.broadcasted_iota(jnp.int32, (H, HC), 1)
    p8 = jnp.where(r8 == c8 // C, 1.0, 0.0)

    hsn = hs_ref[0:N, :]
    h = hsn[:, 0:HC]
    alpha_self = (jnp.dot(hsn, ps, preferred_element_type=jnp.float32)
                  + jnp.dot(ad_ref[0:N, :], p8,
                            preferred_element_type=jnp.float32)
                  + jnp.dot(aself_ref[...], p8,
                            preferred_element_type=jnp.float32))
    ex_s = jnp.exp(_lrelu(alpha_self, 0.2))
    dsum = den_ref[0, 0:N, :] + den_ref[1, 0:N, :]
    den = (jnp.dot(dsum, p8, preferred_element_type=jnp.float32)
           + ex_s + 1e-16)
    num = msg_ref[0, 0:N, :] + msg_ref[1, 0:N, :] + h * ex_s
    out = num / den + bias_ref[...]
    mean = out.mean(axis=0, keepdims=True)
    cen = out - mean
    var = (cen * cen).mean(axis=0, keepdims=True)
    y = cen * jax.lax.rsqrt(var + 1e-5) * gamma_ref[...] + beta_ref[...]
    y_out[...] = _lrelu(y, 0.01)


def _tc_final(msg_p, den_p, hs, ad, aself, bias, gamma, beta):
    return pl.pallas_call(
        _final_body,
        out_shape=jax.ShapeDtypeStruct((N, HC), jnp.float32),
    )(msg_p, den_p, hs, ad, aself,
      bias.reshape(1, HC), gamma.reshape(1, HC), beta.reshape(1, HC))


def kernel(x, edge_idx, edge_attr, W, att_src, att_dst, W_edge, att_edge,
           bias, gamma, beta):
    src = edge_idx[0]
    dst = edge_idx[1]
    eaT = edge_attr.T
    ea0 = eaT[0]
    ea1 = eaT[1]
    hs, ad, m2, aself = _tc_prep(x, W, att_src, att_dst, eaT, W_edge, att_edge)
    msg_p, den_p = _sc_edge_call(src, dst, ea0, ea1, m2, hs, ad)
    return _tc_final(msg_p, den_p, hs, ad, aself, bias, gamma, beta)
